# baseline (speedup 1.0000x reference)
import functools

import jax
import jax.numpy as jnp
from jax import lax
from jax.experimental import pallas as pl
from jax.experimental.pallas import tpu as pltpu
from jax.experimental.pallas import tpu_sc as plsc

N_TASKS = 80
TASK_DIM = 96
BATCH = 16384
OBS_DIM = 512
OUT_DIM = OBS_DIM + TASK_DIM


def _renorm_body(w_ref, out_ref):
    w = w_ref[...]
    ss = jnp.sum(w * w, axis=1, keepdims=True)
    scale = jnp.where(ss > 1.0, lax.rsqrt(ss), 1.0)
    out_ref[...] = w * scale


def _renorm_table(w):
    return pl.pallas_call(
        _renorm_body,
        out_shape=jax.ShapeDtypeStruct((N_TASKS, TASK_DIM), jnp.float32),
    )(w)


_info = plsc.get_sparse_core_info()
_NC = _info.num_cores
_NS = _info.num_subcores
_NW = _NC * _NS
_B_PER_W = BATCH // _NW
_CH = 128


@functools.partial(
    pl.kernel,
    mesh=plsc.VectorSubcoreMesh(core_axis_name="c", subcore_axis_name="s"),
    out_type=jax.ShapeDtypeStruct((BATCH, OUT_DIM), jnp.float32),
    compiler_params=pltpu.CompilerParams(use_tc_tiling_on_sc=False),
    scratch_types=[
        pltpu.VMEM((_B_PER_W,), jnp.int32),
        pltpu.VMEM((_B_PER_W, TASK_DIM), jnp.float32),
        pltpu.VMEM((_CH, OBS_DIM), jnp.float32),
        pltpu.SemaphoreType.DMA,
    ],
)
def _sc_assemble(obs_hbm, task_hbm, table_hbm, out_hbm, idx_v, emb_v, obs_v, sem):
    wid = lax.axis_index("s") * _NC + lax.axis_index("c")
    base = wid * _B_PER_W
    pltpu.sync_copy(task_hbm.at[pl.ds(base, _B_PER_W)], idx_v)
    pltpu.async_copy(table_hbm.at[idx_v], emb_v, sem).wait()
    pltpu.sync_copy(
        emb_v, out_hbm.at[pl.ds(base, _B_PER_W), pl.ds(OBS_DIM, TASK_DIM)]
    )
    pltpu.sync_copy(
        obs_hbm.at[pl.ds(base, _B_PER_W), :],
        out_hbm.at[pl.ds(base, _B_PER_W), pl.ds(0, OBS_DIM)],
    )


def kernel(obs, task, task_emb_weight):
    table_rn = _renorm_table(task_emb_weight)
    return _sc_assemble(obs, task, table_rn)


# speedup vs baseline: 6.9163x; 6.9163x over previous
import functools

import jax
import jax.numpy as jnp
from jax import lax
from jax.experimental import pallas as pl
from jax.experimental.pallas import tpu as pltpu
from jax.experimental.pallas import tpu_sc as plsc

N_TASKS = 80
TASK_DIM = 96
BATCH = 16384
OBS_DIM = 512
OUT_DIM = OBS_DIM + TASK_DIM


def _renorm_body(w_ref, out_ref):
    w = w_ref[...]
    ss = jnp.sum(w * w, axis=1, keepdims=True)
    scale = jnp.where(ss > 1.0, lax.rsqrt(ss), 1.0)
    out_ref[...] = w * scale


def _renorm_table(w):
    return pl.pallas_call(
        _renorm_body,
        out_shape=jax.ShapeDtypeStruct((N_TASKS, TASK_DIM), jnp.float32),
    )(w)


_info = plsc.get_sparse_core_info()
_NC = _info.num_cores
_NS = _info.num_subcores
_NW = _NC * _NS
_B_PER_W = BATCH // _NW
_CH = 32
_NCHUNK = _B_PER_W // _CH
_NBUF = 3


@functools.partial(
    pl.kernel,
    mesh=plsc.VectorSubcoreMesh(core_axis_name="c", subcore_axis_name="s"),
    out_type=jax.ShapeDtypeStruct((BATCH, OUT_DIM), jnp.float32),
    compiler_params=pltpu.CompilerParams(use_tc_tiling_on_sc=False),
    scratch_types=[
        pltpu.VMEM((_B_PER_W,), jnp.int32),
        pltpu.VMEM((_B_PER_W, TASK_DIM), jnp.float32),
    ]
    + [pltpu.VMEM((_CH, OBS_DIM), jnp.float32)] * _NBUF
    + [pltpu.SemaphoreType.DMA] * (2 + 2 * _NBUF),
)
def _sc_assemble(
    obs_hbm, task_hbm, table_hbm, out_hbm, idx_v, emb_v, *rest
):
    bufs = rest[:_NBUF]
    sem_g, sem_e = rest[_NBUF], rest[_NBUF + 1]
    rsems = rest[_NBUF + 2 : 2 * _NBUF + 2]
    wsems = rest[2 * _NBUF + 2 :]
    wid = lax.axis_index("s") * _NC + lax.axis_index("c")
    base = wid * _B_PER_W
    pltpu.sync_copy(task_hbm.at[pl.ds(base, _B_PER_W)], idx_v)
    gather = pltpu.async_copy(table_hbm.at[idx_v], emb_v, sem_g)
    emb_write = None
    reads = [None] * _NCHUNK
    writes = [None] * _NCHUNK
    for c in range(_NCHUNK + 1):
        if c < _NCHUNK:
            b = c % _NBUF
            if c >= _NBUF:
                writes[c - _NBUF].wait()
            reads[c] = pltpu.async_copy(
                obs_hbm.at[pl.ds(base + c * _CH, _CH), :], bufs[b], rsems[b]
            )
        if c == 2:
            gather.wait()
            emb_write = pltpu.async_copy(
                emb_v,
                out_hbm.at[pl.ds(base, _B_PER_W), pl.ds(OBS_DIM, TASK_DIM)],
                sem_e,
            )
        if c >= 1:
            c2 = c - 1
            reads[c2].wait()
            writes[c2] = pltpu.async_copy(
                bufs[c2 % _NBUF],
                out_hbm.at[pl.ds(base + c2 * _CH, _CH), pl.ds(0, OBS_DIM)],
                wsems[c2 % _NBUF],
            )
    for c in range(_NCHUNK - _NBUF, _NCHUNK):
        writes[c].wait()
    emb_write.wait()


def kernel(obs, task, task_emb_weight):
    table_rn = _renorm_table(task_emb_weight)
    return _sc_assemble(obs, task, table_rn)


# speedup vs baseline: 10.3525x; 1.4968x over previous
import functools

import jax
import jax.numpy as jnp
from jax import lax
from jax.experimental import pallas as pl
from jax.experimental.pallas import tpu as pltpu
from jax.experimental.pallas import tpu_sc as plsc

N_TASKS = 80
TASK_DIM = 96
BATCH = 16384
OBS_DIM = 512
OUT_DIM = OBS_DIM + TASK_DIM
_PAD_DIM = 128
_L = 16


def _renorm_body(w_ref, out_ref):
    w = w_ref[...]
    ss = jnp.sum(w * w, axis=1, keepdims=True)
    scale = jnp.where(ss > 1.0, lax.rsqrt(ss), 1.0)
    out_ref[...] = jnp.concatenate(
        [w * scale, jnp.zeros((N_TASKS, _PAD_DIM - TASK_DIM), jnp.float32)], axis=1
    )


def _renorm_table(w):
    return pl.pallas_call(
        _renorm_body,
        out_shape=jax.ShapeDtypeStruct((N_TASKS, _PAD_DIM), jnp.float32),
    )(w)


_info = plsc.get_sparse_core_info()
_NC = _info.num_cores
_NS = _info.num_subcores
_NW = _NC * _NS
_B_PER_W = BATCH // _NW
_CH = 32
_NCHUNK = _B_PER_W // _CH
_NBUF = 4
_LAG = 2


@functools.partial(
    pl.kernel,
    mesh=plsc.VectorSubcoreMesh(core_axis_name="c", subcore_axis_name="s"),
    out_type=jax.ShapeDtypeStruct((BATCH, OUT_DIM), jnp.float32),
    scratch_types=[pltpu.VMEM((_B_PER_W,), jnp.int32)]
    + [pltpu.VMEM((_CH, OUT_DIM), jnp.float32)] * _NBUF
    + [pltpu.VMEM((_CH, _PAD_DIM), jnp.float32)] * _NBUF
    + [pltpu.SemaphoreType.DMA] * (3 * _NBUF),
)
def _sc_assemble(obs_hbm, task_hbm, table_hbm, out_hbm, idx_v, *rest):
    stg = rest[:_NBUF]
    emb = rest[_NBUF : 2 * _NBUF]
    osems = rest[2 * _NBUF : 3 * _NBUF]
    gsems = rest[3 * _NBUF : 4 * _NBUF]
    wsems = rest[4 * _NBUF :]
    wid = lax.axis_index("s") * _NC + lax.axis_index("c")
    base = wid * _B_PER_W
    pltpu.sync_copy(task_hbm.at[pl.ds(base, _B_PER_W)], idx_v)

    reads = [None] * _NCHUNK
    gathers = [None] * _NCHUNK
    writes = [None] * _NCHUNK

    def start(c):
        b = c % _NBUF
        reads[c] = pltpu.async_copy(
            obs_hbm.at[pl.ds(base + c * _CH, _CH), :],
            stg[b].at[:, pl.ds(0, OBS_DIM)],
            osems[b],
        )
        gathers[c] = pltpu.async_copy(
            table_hbm.at[idx_v.at[pl.ds(c * _CH, _CH)]], emb[b], gsems[b]
        )

    def process(c):
        b = c % _NBUF
        gathers[c].wait()

        def marshal(r, carry):
            for k in range(TASK_DIM // _L):
                stg[b][r, pl.ds(OBS_DIM + k * _L, _L)] = emb[b][r, pl.ds(k * _L, _L)]
            return carry

        lax.fori_loop(0, _CH, marshal, 0)
        reads[c].wait()
        writes[c] = pltpu.async_copy(
            stg[b], out_hbm.at[pl.ds(base + c * _CH, _CH), :], wsems[b]
        )

    for c in range(_NCHUNK):
        if c >= _NBUF:
            writes[c - _NBUF].wait()
        start(c)
        if c >= _LAG:
            process(c - _LAG)
    for c in range(_NCHUNK - _LAG, _NCHUNK):
        process(c)
    for c in range(_NCHUNK - _NBUF, _NCHUNK):
        writes[c].wait()


def kernel(obs, task, task_emb_weight):
    table_rn = _renorm_table(task_emb_weight)
    return _sc_assemble(obs, task, table_rn)
